# Initial kernel scaffold; baseline (speedup 1.0000x reference)
import jax
import jax.numpy as jnp
from jax.experimental import pallas as pl


def kernel(x, edge_index, edge_attr, batchh, emb, l1_msg_w, l1_msg_b, l1_self_w, l1_self_b, l1_edge_w, l1_edge_b, l1_att, l2_msg_w, l2_msg_b, l2_self_w, l2_self_b, l2_edge_w, l2_edge_b, l2_att, cls_w1, cls_b1, cls_w2, cls_b2):
    raise NotImplementedError("write your pallas kernel here")


# speedup vs baseline: 6.2208x; 6.2208x over previous
import functools

import jax
import jax.numpy as jnp
from jax import lax
from jax.experimental import pallas as pl
from jax.experimental.pallas import tpu as pltpu
from jax.experimental.pallas import tpu_sc as plsc

N = 50000
E = 800000
EMB = 16
H1 = 32
H2 = 64
EDIM = 16
OUT = 8
NGRAPH = 128

NC = 2
NS = 16
HALF = N // NC
ZSTRIPE = 1568
ACC_ROWS = NS * ZSTRIPE
DSTRIPE = 1560
CH = E // NS
K = 80
ITERS = CH // K


def _sc_layer(din):
    gw = din + 16
    roww = din + 32

    def body(g_hbm, src_hbm, dst_hbm, ea_hbm, se_hbm, z_hbm, out_hbm,
             accum, srcb, dstb, sidx, grows, eab, seb, outb, sem):
        c = lax.axis_index("c")
        s = lax.axis_index("s")
        base = c * HALF
        pltpu.sync_copy(z_hbm, accum.at[pl.ds(s * ZSTRIPE, ZSTRIPE)])
        plsc.subcore_barrier()

        ebase = s * CH

        def it(i, carry):
            off = ebase + i * K
            cp1 = pltpu.async_copy(src_hbm.at[pl.ds(off, K)], srcb, sem)
            cp2 = pltpu.async_copy(dst_hbm.at[pl.ds(off, K)], dstb, sem)
            cp3 = pltpu.async_copy(ea_hbm.at[pl.ds(off, K)], eab, sem)
            cp4 = pltpu.async_copy(se_hbm.at[pl.ds(off, K)], seb, sem)
            cp1.wait(); cp2.wait(); cp3.wait(); cp4.wait()
            pltpu.async_copy(g_hbm.at[srcb], grows, sem).wait()
            lane = lax.iota(jnp.int32, 16)
            for gi in range(K // 16):
                sl = pl.ds(gi * 16, 16)
                sev = seb[sl]
                dv = dstb[sl] - base
                inb = (dv >= 0) & (dv < HALF)
                sidx[sl] = jnp.where(inb, dv, HALF)
                for j in range(16):
                    e = gi * 16 + j
                    sn = grows[e, pl.ds(din, 16)]
                    alpha = (jnp.full((16,), sn[0], jnp.float32)
                             + jnp.full((16,), sev[j], jnp.float32))
                    alpha = jnp.where(alpha >= 0.0, alpha, alpha * 0.2)
                    ex = jnp.exp(alpha)
                    for d0 in range(0, din, 16):
                        outb[e, pl.ds(d0, 16)] = ex * grows[e, pl.ds(d0, 16)]
                    outb[e, pl.ds(din, EDIM)] = ex * eab[e, :]
                    tail = jnp.where(lane == 0, ex,
                                     jnp.where(lane == 1, 1.0, 0.0))
                    outb[e, pl.ds(din + EDIM, 16)] = tail
            pltpu.sync_copy(outb, accum.at[sidx], add=True)
            return carry

        lax.fori_loop(0, ITERS, it, 0)
        plsc.subcore_barrier()
        pltpu.sync_copy(accum.at[pl.ds(s * DSTRIPE, DSTRIPE)],
                        out_hbm.at[pl.ds(base + s * DSTRIPE, DSTRIPE)])

        @pl.when(s == 0)
        def _():
            pltpu.sync_copy(accum.at[pl.ds(NS * DSTRIPE, HALF - NS * DSTRIPE)],
                            out_hbm.at[pl.ds(base + NS * DSTRIPE,
                                             HALF - NS * DSTRIPE)])

    mesh = plsc.VectorSubcoreMesh(core_axis_name="c", subcore_axis_name="s")
    return pl.kernel(
        body,
        out_type=jax.ShapeDtypeStruct((N, roww), jnp.float32),
        mesh=mesh,
        compiler_params=pltpu.CompilerParams(use_tc_tiling_on_sc=False),
        scratch_types=[
            pltpu.VMEM_SHARED((ACC_ROWS, roww), jnp.float32),
            pltpu.VMEM((K,), jnp.int32),
            pltpu.VMEM((K,), jnp.int32),
            pltpu.VMEM((K,), jnp.int32),
            pltpu.VMEM((K, gw), jnp.float32),
            pltpu.VMEM((K, EDIM), jnp.float32),
            pltpu.VMEM((K,), jnp.float32),
            pltpu.VMEM((K, roww), jnp.float32),
            pltpu.SemaphoreType.DMA,
        ],
    )


EB = 6400
NB = 2000


def _edge_prep_body(ea_ref, wv_ref, out_ref):
    out_ref[...] = jnp.dot(ea_ref[...], wv_ref[...],
                           preferred_element_type=jnp.float32)


def _edge_prep(edge_attr, wv):
    return pl.pallas_call(
        _edge_prep_body,
        grid=(E // EB,),
        in_specs=[
            pl.BlockSpec((EB, EDIM), lambda i: (i, 0)),
            pl.BlockSpec((EDIM, 2), lambda i: (0, 0)),
        ],
        out_specs=pl.BlockSpec((EB, 2), lambda i: (i, 0)),
        out_shape=jax.ShapeDtypeStruct((E, 2), jnp.float32),
    )(edge_attr, wv)


def _node_prep_body(h_ref, wn_ref, c_ref, out_ref):
    h = h_ref[...]
    col = jnp.dot(h, wn_ref[...], preferred_element_type=jnp.float32)
    pad = jnp.zeros((h.shape[0], 15), jnp.float32)
    out_ref[...] = jnp.concatenate([h, col + c_ref[0, 0], pad], axis=1)


def _node_prep(h, wn, c):
    din = h.shape[1]
    return pl.pallas_call(
        _node_prep_body,
        grid=(N // NB,),
        in_specs=[
            pl.BlockSpec((NB, din), lambda i: (i, 0)),
            pl.BlockSpec((din, 1), lambda i: (0, 0)),
            pl.BlockSpec((1, 1), lambda i: (0, 0)),
        ],
        out_specs=pl.BlockSpec((NB, din + 16), lambda i: (i, 0)),
        out_shape=jax.ShapeDtypeStruct((N, din + 16), jnp.float32),
    )(h, wn, c)


def _post_body(din, hout, next_cols, acc_ref, g_ref, mw_ref, ew_ref, bs_ref,
               sw_ref, sb_ref, wn_ref, c_ref, out_ref):
    acc = acc_ref[...]
    sh = acc[:, :din]
    sea = acc[:, din:din + EDIM]
    sex = acc[:, din + EDIM]
    cnt = acc[:, din + EDIM + 1]
    num = (jnp.dot(sh, mw_ref[...], preferred_element_type=jnp.float32)
           + jnp.dot(sea, ew_ref[...], preferred_element_type=jnp.float32)
           + sex[:, None] * bs_ref[...])
    agg = num / (sex + 1e-16)[:, None] / jnp.maximum(cnt, 1.0)[:, None]
    h = g_ref[...][:, :din]
    hn = jax.nn.relu(
        agg + jnp.dot(h, sw_ref[...], preferred_element_type=jnp.float32)
        + sb_ref[...])
    if next_cols:
        col = jnp.dot(hn, wn_ref[...], preferred_element_type=jnp.float32)
        pad = jnp.zeros((hn.shape[0], 15), jnp.float32)
        out_ref[...] = jnp.concatenate([hn, col + c_ref[0, 0], pad], axis=1)
    else:
        out_ref[...] = hn


def _post(acc, g, mw, ew, bsum, sw, sb, wn, c, next_cols):
    din = mw.shape[0]
    hout = mw.shape[1]
    ow = hout + (16 if next_cols else 0)
    return pl.pallas_call(
        functools.partial(_post_body, din, hout, next_cols),
        grid=(N // NB,),
        in_specs=[
            pl.BlockSpec((NB, acc.shape[1]), lambda i: (i, 0)),
            pl.BlockSpec((NB, g.shape[1]), lambda i: (i, 0)),
            pl.BlockSpec((din, hout), lambda i: (0, 0)),
            pl.BlockSpec((EDIM, hout), lambda i: (0, 0)),
            pl.BlockSpec((1, hout), lambda i: (0, 0)),
            pl.BlockSpec((din, hout), lambda i: (0, 0)),
            pl.BlockSpec((1, hout), lambda i: (0, 0)),
            pl.BlockSpec((hout, 1), lambda i: (0, 0)),
            pl.BlockSpec((1, 1), lambda i: (0, 0)),
        ],
        out_specs=pl.BlockSpec((NB, ow), lambda i: (i, 0)),
        out_shape=jax.ShapeDtypeStruct((N, ow), jnp.float32),
    )(acc, g, mw, ew, bsum, sw, sb, wn, c)


def _pool_body(h_ref, b_ref, w1_ref, b1_ref, w2_ref, b2_ref, out_ref, ps_ref):
    i = pl.program_id(0)

    @pl.when(i == 0)
    def _():
        ps_ref[...] = jnp.zeros_like(ps_ref)

    h = h_ref[...]
    bb = b_ref[...]
    onehot = (bb == lax.broadcasted_iota(jnp.int32, (1, NGRAPH), 1)
              ).astype(jnp.float32)
    hcat = jnp.concatenate(
        [h, jnp.ones((h.shape[0], 1), jnp.float32)], axis=1)
    ps_ref[...] += jnp.dot(onehot.T, hcat,
                           preferred_element_type=jnp.float32)
    ps = ps_ref[...]
    pooled = ps[:, :H2] / jnp.maximum(ps[:, H2:H2 + 1], 1.0)
    hidden = jax.nn.relu(
        jnp.dot(pooled, w1_ref[...], preferred_element_type=jnp.float32)
        + b1_ref[...])
    out_ref[...] = (jnp.dot(hidden, w2_ref[...],
                            preferred_element_type=jnp.float32) + b2_ref[...])


def _pool_cls(h, batch2d, w1, b1, w2, b2):
    return pl.pallas_call(
        _pool_body,
        grid=(N // NB,),
        in_specs=[
            pl.BlockSpec((NB, H2), lambda i: (i, 0)),
            pl.BlockSpec((NB, 1), lambda i: (i, 0)),
            pl.BlockSpec((H2, H2), lambda i: (0, 0)),
            pl.BlockSpec((1, H2), lambda i: (0, 0)),
            pl.BlockSpec((H2, OUT), lambda i: (0, 0)),
            pl.BlockSpec((1, OUT), lambda i: (0, 0)),
        ],
        out_specs=pl.BlockSpec((NGRAPH, OUT), lambda i: (0, 0)),
        out_shape=jax.ShapeDtypeStruct((NGRAPH, OUT), jnp.float32),
        scratch_shapes=[pltpu.VMEM((NGRAPH, H2 + 1), jnp.float32)],
    )(h, batch2d, w1, b1, w2, b2)


def kernel(x, edge_index, edge_attr, batchh, emb,
           l1_msg_w, l1_msg_b, l1_self_w, l1_self_b, l1_edge_w, l1_edge_b,
           l1_att,
           l2_msg_w, l2_msg_b, l2_self_w, l2_self_b, l2_edge_w, l2_edge_b,
           l2_att,
           cls_w1, cls_b1, cls_w2, cls_b2):
    h0 = jnp.take(emb, x, axis=0)
    srcs = edge_index[0]
    dsts = edge_index[1]

    a1 = l1_att[0]
    a2 = l2_att[0]
    wv = jnp.stack([l1_edge_w @ a1, l2_edge_w @ a2], axis=1)
    wn1 = (l1_msg_w @ a1)[:, None]
    c1 = jnp.reshape((l1_msg_b + l1_edge_b) @ a1, (1, 1))
    wn2 = (l2_msg_w @ a2)[:, None]
    c2 = jnp.reshape((l2_msg_b + l2_edge_b) @ a2, (1, 1))
    bs1 = (l1_msg_b + l1_edge_b)[None, :]
    bs2 = (l2_msg_b + l2_edge_b)[None, :]

    se_both = _edge_prep(edge_attr, wv)
    se1 = se_both[:, 0] + 0.0
    se2 = se_both[:, 1] + 0.0

    g1 = _node_prep(h0, wn1, c1)
    z1 = jnp.zeros((ZSTRIPE, EMB + 32), jnp.float32)
    acc1 = _sc_layer(EMB)(g1, srcs, dsts, edge_attr, se1, z1)
    g2 = _post(acc1, g1, l1_msg_w, l1_edge_w, bs1, l1_self_w,
               l1_self_b[None, :], wn2, c2, True)

    z2 = jnp.zeros((ZSTRIPE, H1 + 32), jnp.float32)
    acc2 = _sc_layer(H1)(g2, srcs, dsts, edge_attr, se2, z2)
    h2 = _post(acc2, g2, l2_msg_w, l2_edge_w, bs2, l2_self_w,
               l2_self_b[None, :], wn2, c2, False)

    return _pool_cls(h2, batchh[:, None], cls_w1, cls_b1[None, :],
                     cls_w2, cls_b2[None, :])


# speedup vs baseline: 6.2287x; 1.0013x over previous
import functools

import jax
import jax.numpy as jnp
from jax import lax
from jax.experimental import pallas as pl
from jax.experimental.pallas import tpu as pltpu
from jax.experimental.pallas import tpu_sc as plsc

N = 50000
E = 800000
EMB = 16
H1 = 32
H2 = 64
EDIM = 16
OUT = 8
NGRAPH = 128

NC = 2
NS = 16
HALF = N // NC
ZSTRIPE = 1568
ACC_ROWS = NS * ZSTRIPE
DSTRIPE = 1560
CH = E // NS
KB = 80
PAIRS = CH // (2 * KB)
EPI_OFF = CH - KB
EPI_SKIP = (2 * KB * PAIRS) - EPI_OFF


def _sc_layer(din):
    gw = din + 16
    roww = din + 32

    def body(g_hbm, src_hbm, dst_hbm, ea_hbm, se_hbm, z_hbm, out_hbm,
             accum, srcA, srcB, dstA, dstB, sidxA, sidxB, sidx80,
             growsA, growsB, eaA, eaB, seA, seB, outA, outB,
             semA, semB):
        c = lax.axis_index("c")
        s = lax.axis_index("s")
        base = c * HALF
        pltpu.sync_copy(z_hbm, accum.at[pl.ds(s * ZSTRIPE, ZSTRIPE)])
        plsc.subcore_barrier()

        ebase = s * CH
        lane = lax.iota(jnp.int32, 16)

        def compute_half(grows, eab, seb, dstb, outb, sidx_ref, groups, gskip):
            for gi in groups:
                sl = pl.ds(gi * 16, 16)
                sev = seb[sl]
                dv = dstb[sl] - base
                inb = (dv >= 0) & (dv < HALF)
                sidx_ref[pl.ds((gi - gskip) * 16, 16)] = (
                    jnp.where(inb, dv, HALF))
                for j in range(16):
                    e = gi * 16 + j
                    sn = grows[e, pl.ds(din, 16)]
                    alpha = (jnp.full((16,), sn[0], jnp.float32)
                             + jnp.full((16,), sev[j], jnp.float32))
                    alpha = jnp.where(alpha >= 0.0, alpha, alpha * 0.2)
                    ex = jnp.exp(alpha)
                    for d0 in range(0, din, 16):
                        outb[e, pl.ds(d0, 16)] = ex * grows[e, pl.ds(d0, 16)]
                    outb[e, pl.ds(din, EDIM)] = ex * eab[e, :]
                    tail = jnp.where(lane == 0, ex,
                                     jnp.where(lane == 1, 1.0, 0.0))
                    outb[e, pl.ds(din + EDIM, 16)] = tail

        def it(i, carry):
            offA = ebase + i * (2 * KB)
            offB = offA + KB
            a1 = pltpu.async_copy(src_hbm.at[pl.ds(offA, KB)], srcA, semA)
            b1 = pltpu.async_copy(src_hbm.at[pl.ds(offB, KB)], srcB, semB)
            a2 = pltpu.async_copy(dst_hbm.at[pl.ds(offA, KB)], dstA, semA)
            a3 = pltpu.async_copy(ea_hbm.at[pl.ds(offA, KB)], eaA, semA)
            a4 = pltpu.async_copy(se_hbm.at[pl.ds(offA, KB)], seA, semA)
            b2 = pltpu.async_copy(dst_hbm.at[pl.ds(offB, KB)], dstB, semB)
            b3 = pltpu.async_copy(ea_hbm.at[pl.ds(offB, KB)], eaB, semB)
            b4 = pltpu.async_copy(se_hbm.at[pl.ds(offB, KB)], seB, semB)
            a1.wait(); a2.wait(); a3.wait(); a4.wait()
            gA = pltpu.async_copy(g_hbm.at[srcA], growsA, semA)
            b1.wait(); b2.wait(); b3.wait(); b4.wait()
            gB = pltpu.async_copy(g_hbm.at[srcB], growsB, semB)
            gA.wait()
            compute_half(growsA, eaA, seA, dstA, outA, sidxA,
                         range(KB // 16), 0)
            pltpu.sync_copy(outA, accum.at[sidxA], add=True)
            gB.wait()
            compute_half(growsB, eaB, seB, dstB, outB, sidxB,
                         range(KB // 16), 0)
            pltpu.sync_copy(outB, accum.at[sidxB], add=True)
            return carry

        lax.fori_loop(0, PAIRS, it, 0)
        offE = ebase + EPI_OFF
        e1 = pltpu.async_copy(src_hbm.at[pl.ds(offE, KB)], srcA, semA)
        e2 = pltpu.async_copy(dst_hbm.at[pl.ds(offE, KB)], dstA, semA)
        e3 = pltpu.async_copy(ea_hbm.at[pl.ds(offE, KB)], eaA, semA)
        e4 = pltpu.async_copy(se_hbm.at[pl.ds(offE, KB)], seA, semA)
        e1.wait(); e2.wait(); e3.wait(); e4.wait()
        gE = pltpu.async_copy(g_hbm.at[srcA], growsA, semA)
        gE.wait()
        compute_half(growsA, eaA, seA, dstA, outA, sidx80,
                     range(EPI_SKIP // 16, KB // 16), EPI_SKIP // 16)
        if EPI_SKIP == 0:
            pltpu.sync_copy(outA, accum.at[sidx80], add=True)
        else:
            pltpu.sync_copy(outA.at[pl.ds(EPI_SKIP, KB - EPI_SKIP)],
                            accum.at[sidx80], add=True)
        plsc.subcore_barrier()
        pltpu.sync_copy(accum.at[pl.ds(s * DSTRIPE, DSTRIPE)],
                        out_hbm.at[pl.ds(base + s * DSTRIPE, DSTRIPE)])

        @pl.when(s == 0)
        def _():
            pltpu.sync_copy(accum.at[pl.ds(NS * DSTRIPE, HALF - NS * DSTRIPE)],
                            out_hbm.at[pl.ds(base + NS * DSTRIPE,
                                             HALF - NS * DSTRIPE)])

    mesh = plsc.VectorSubcoreMesh(core_axis_name="c", subcore_axis_name="s")
    return pl.kernel(
        body,
        out_type=jax.ShapeDtypeStruct((N, roww), jnp.float32),
        mesh=mesh,
        compiler_params=pltpu.CompilerParams(use_tc_tiling_on_sc=False),
        scratch_types=[
            pltpu.VMEM_SHARED((ACC_ROWS, roww), jnp.float32),
            pltpu.VMEM((KB,), jnp.int32),
            pltpu.VMEM((KB,), jnp.int32),
            pltpu.VMEM((KB,), jnp.int32),
            pltpu.VMEM((KB,), jnp.int32),
            pltpu.VMEM((KB,), jnp.int32),
            pltpu.VMEM((KB,), jnp.int32),
            pltpu.VMEM((KB - EPI_SKIP,), jnp.int32),
            pltpu.VMEM((KB, gw), jnp.float32),
            pltpu.VMEM((KB, gw), jnp.float32),
            pltpu.VMEM((KB, EDIM), jnp.float32),
            pltpu.VMEM((KB, EDIM), jnp.float32),
            pltpu.VMEM((KB,), jnp.float32),
            pltpu.VMEM((KB,), jnp.float32),
            pltpu.VMEM((KB, roww), jnp.float32),
            pltpu.VMEM((KB, roww), jnp.float32),
            pltpu.SemaphoreType.DMA,
            pltpu.SemaphoreType.DMA,
        ],
    )


EB = 6400
NB = 2000


def _edge_prep_body(ea_ref, wv_ref, out_ref):
    out_ref[...] = jnp.dot(ea_ref[...], wv_ref[...],
                           preferred_element_type=jnp.float32)


def _edge_prep(edge_attr, wv):
    return pl.pallas_call(
        _edge_prep_body,
        grid=(E // EB,),
        in_specs=[
            pl.BlockSpec((EB, EDIM), lambda i: (i, 0)),
            pl.BlockSpec((EDIM, 2), lambda i: (0, 0)),
        ],
        out_specs=pl.BlockSpec((EB, 2), lambda i: (i, 0)),
        out_shape=jax.ShapeDtypeStruct((E, 2), jnp.float32),
    )(edge_attr, wv)


def _node_prep_body(h_ref, wn_ref, c_ref, out_ref):
    h = h_ref[...]
    col = jnp.dot(h, wn_ref[...], preferred_element_type=jnp.float32)
    pad = jnp.zeros((h.shape[0], 15), jnp.float32)
    out_ref[...] = jnp.concatenate([h, col + c_ref[0, 0], pad], axis=1)


def _node_prep(h, wn, c):
    din = h.shape[1]
    return pl.pallas_call(
        _node_prep_body,
        grid=(N // NB,),
        in_specs=[
            pl.BlockSpec((NB, din), lambda i: (i, 0)),
            pl.BlockSpec((din, 1), lambda i: (0, 0)),
            pl.BlockSpec((1, 1), lambda i: (0, 0)),
        ],
        out_specs=pl.BlockSpec((NB, din + 16), lambda i: (i, 0)),
        out_shape=jax.ShapeDtypeStruct((N, din + 16), jnp.float32),
    )(h, wn, c)


def _post_body(din, hout, next_cols, acc_ref, g_ref, mw_ref, ew_ref, bs_ref,
               sw_ref, sb_ref, wn_ref, c_ref, out_ref):
    acc = acc_ref[...]
    sh = acc[:, :din]
    sea = acc[:, din:din + EDIM]
    sex = acc[:, din + EDIM]
    cnt = acc[:, din + EDIM + 1]
    num = (jnp.dot(sh, mw_ref[...], preferred_element_type=jnp.float32)
           + jnp.dot(sea, ew_ref[...], preferred_element_type=jnp.float32)
           + sex[:, None] * bs_ref[...])
    agg = num / (sex + 1e-16)[:, None] / jnp.maximum(cnt, 1.0)[:, None]
    h = g_ref[...][:, :din]
    hn = jax.nn.relu(
        agg + jnp.dot(h, sw_ref[...], preferred_element_type=jnp.float32)
        + sb_ref[...])
    if next_cols:
        col = jnp.dot(hn, wn_ref[...], preferred_element_type=jnp.float32)
        pad = jnp.zeros((hn.shape[0], 15), jnp.float32)
        out_ref[...] = jnp.concatenate([hn, col + c_ref[0, 0], pad], axis=1)
    else:
        out_ref[...] = hn


def _post(acc, g, mw, ew, bsum, sw, sb, wn, c, next_cols):
    din = mw.shape[0]
    hout = mw.shape[1]
    ow = hout + (16 if next_cols else 0)
    return pl.pallas_call(
        functools.partial(_post_body, din, hout, next_cols),
        grid=(N // NB,),
        in_specs=[
            pl.BlockSpec((NB, acc.shape[1]), lambda i: (i, 0)),
            pl.BlockSpec((NB, g.shape[1]), lambda i: (i, 0)),
            pl.BlockSpec((din, hout), lambda i: (0, 0)),
            pl.BlockSpec((EDIM, hout), lambda i: (0, 0)),
            pl.BlockSpec((1, hout), lambda i: (0, 0)),
            pl.BlockSpec((din, hout), lambda i: (0, 0)),
            pl.BlockSpec((1, hout), lambda i: (0, 0)),
            pl.BlockSpec((hout, 1), lambda i: (0, 0)),
            pl.BlockSpec((1, 1), lambda i: (0, 0)),
        ],
        out_specs=pl.BlockSpec((NB, ow), lambda i: (i, 0)),
        out_shape=jax.ShapeDtypeStruct((N, ow), jnp.float32),
    )(acc, g, mw, ew, bsum, sw, sb, wn, c)


def _pool_body(h_ref, b_ref, w1_ref, b1_ref, w2_ref, b2_ref, out_ref, ps_ref):
    i = pl.program_id(0)

    @pl.when(i == 0)
    def _():
        ps_ref[...] = jnp.zeros_like(ps_ref)

    h = h_ref[...]
    bb = b_ref[...]
    onehot = (bb == lax.broadcasted_iota(jnp.int32, (1, NGRAPH), 1)
              ).astype(jnp.float32)
    hcat = jnp.concatenate(
        [h, jnp.ones((h.shape[0], 1), jnp.float32)], axis=1)
    ps_ref[...] += jnp.dot(onehot.T, hcat,
                           preferred_element_type=jnp.float32)
    ps = ps_ref[...]
    pooled = ps[:, :H2] / jnp.maximum(ps[:, H2:H2 + 1], 1.0)
    hidden = jax.nn.relu(
        jnp.dot(pooled, w1_ref[...], preferred_element_type=jnp.float32)
        + b1_ref[...])
    out_ref[...] = (jnp.dot(hidden, w2_ref[...],
                            preferred_element_type=jnp.float32) + b2_ref[...])


def _pool_cls(h, batch2d, w1, b1, w2, b2):
    return pl.pallas_call(
        _pool_body,
        grid=(N // NB,),
        in_specs=[
            pl.BlockSpec((NB, H2), lambda i: (i, 0)),
            pl.BlockSpec((NB, 1), lambda i: (i, 0)),
            pl.BlockSpec((H2, H2), lambda i: (0, 0)),
            pl.BlockSpec((1, H2), lambda i: (0, 0)),
            pl.BlockSpec((H2, OUT), lambda i: (0, 0)),
            pl.BlockSpec((1, OUT), lambda i: (0, 0)),
        ],
        out_specs=pl.BlockSpec((NGRAPH, OUT), lambda i: (0, 0)),
        out_shape=jax.ShapeDtypeStruct((NGRAPH, OUT), jnp.float32),
        scratch_shapes=[pltpu.VMEM((NGRAPH, H2 + 1), jnp.float32)],
    )(h, batch2d, w1, b1, w2, b2)


def kernel(x, edge_index, edge_attr, batchh, emb,
           l1_msg_w, l1_msg_b, l1_self_w, l1_self_b, l1_edge_w, l1_edge_b,
           l1_att,
           l2_msg_w, l2_msg_b, l2_self_w, l2_self_b, l2_edge_w, l2_edge_b,
           l2_att,
           cls_w1, cls_b1, cls_w2, cls_b2):
    h0 = jnp.take(emb, x, axis=0)
    srcs = edge_index[0]
    dsts = edge_index[1]

    a1 = l1_att[0]
    a2 = l2_att[0]
    wv = jnp.stack([l1_edge_w @ a1, l2_edge_w @ a2], axis=1)
    wn1 = (l1_msg_w @ a1)[:, None]
    c1 = jnp.reshape((l1_msg_b + l1_edge_b) @ a1, (1, 1))
    wn2 = (l2_msg_w @ a2)[:, None]
    c2 = jnp.reshape((l2_msg_b + l2_edge_b) @ a2, (1, 1))
    bs1 = (l1_msg_b + l1_edge_b)[None, :]
    bs2 = (l2_msg_b + l2_edge_b)[None, :]

    se_both = _edge_prep(edge_attr, wv)
    se1 = se_both[:, 0] + 0.0
    se2 = se_both[:, 1] + 0.0

    g1 = _node_prep(h0, wn1, c1)
    z1 = jnp.zeros((ZSTRIPE, EMB + 32), jnp.float32)
    acc1 = _sc_layer(EMB)(g1, srcs, dsts, edge_attr, se1, z1)
    g2 = _post(acc1, g1, l1_msg_w, l1_edge_w, bs1, l1_self_w,
               l1_self_b[None, :], wn2, c2, True)

    z2 = jnp.zeros((ZSTRIPE, H1 + 32), jnp.float32)
    acc2 = _sc_layer(H1)(g2, srcs, dsts, edge_attr, se2, z2)
    h2 = _post(acc2, g2, l2_msg_w, l2_edge_w, bs2, l2_self_w,
               l2_self_b[None, :], wn2, c2, False)

    return _pool_cls(h2, batchh[:, None], cls_w1, cls_b1[None, :],
                     cls_w2, cls_b2[None, :])


# speedup vs baseline: 7.4431x; 1.1950x over previous
import functools

import jax
import jax.numpy as jnp
from jax import lax
from jax.experimental import pallas as pl
from jax.experimental.pallas import tpu as pltpu
from jax.experimental.pallas import tpu_sc as plsc

N = 50000
E = 800000
EMB = 16
H1 = 32
H2 = 64
EDIM = 16
OUT = 8
NGRAPH = 128

NC = 2
NS = 16
HALF = N // NC
ZSTRIPE = 1568
ACC_ROWS = NS * ZSTRIPE
DSTRIPE = 1560
CH = E // NS
KB = 80
PAIRS = CH // (2 * KB)
EPI_OFF = CH - KB
EPI_SKIP = (2 * KB * PAIRS) - EPI_OFF


def _sc_layer(din):
    gw = din + 16
    roww = din + 32

    def body(g_hbm, src_hbm, dst_hbm, ea_hbm, se_hbm, z_hbm, out_hbm,
             accum, srcA, srcB, dstA, dstB, sidxA, sidxB, sidx80,
             growsA, growsB, eaA, eaB, seA, seB, outA, outB,
             semA, semB, semSA, semSB):
        c = lax.axis_index("c")
        s = lax.axis_index("s")
        base = c * HALF
        pltpu.sync_copy(z_hbm, accum.at[pl.ds(s * ZSTRIPE, ZSTRIPE)])
        plsc.subcore_barrier()

        ebase = s * CH
        lane = lax.iota(jnp.int32, 16)

        def compute_half(grows, eab, seb, dstb, outb, sidx_ref, groups, gskip):
            for gi in groups:
                sl = pl.ds(gi * 16, 16)
                sev = seb[sl]
                dv = dstb[sl] - base
                inb = (dv >= 0) & (dv < HALF)
                sidx_ref[pl.ds((gi - gskip) * 16, 16)] = (
                    jnp.where(inb, dv, HALF))
                for j in range(16):
                    e = gi * 16 + j
                    sn = grows[e, pl.ds(din, 16)]
                    alpha = (jnp.full((16,), sn[0], jnp.float32)
                             + jnp.full((16,), sev[j], jnp.float32))
                    alpha = jnp.where(alpha >= 0.0, alpha, alpha * 0.2)
                    ex = jnp.exp(alpha)
                    for d0 in range(0, din, 16):
                        outb[e, pl.ds(d0, 16)] = ex * grows[e, pl.ds(d0, 16)]
                    outb[e, pl.ds(din, EDIM)] = ex * eab[e, :]
                    tail = jnp.where(lane == 0, ex,
                                     jnp.where(lane == 1, 1.0, 0.0))
                    outb[e, pl.ds(din + EDIM, 16)] = tail

        def issue_a(off):
            pltpu.async_copy(src_hbm.at[pl.ds(off, KB)], srcA, semA)
            pltpu.async_copy(dst_hbm.at[pl.ds(off, KB)], dstA, semA)
            pltpu.async_copy(ea_hbm.at[pl.ds(off, KB)], eaA, semA)
            pltpu.async_copy(se_hbm.at[pl.ds(off, KB)], seA, semA)

        def drain_a(off):
            pltpu.make_async_copy(src_hbm.at[pl.ds(off, KB)], srcA,
                                  semA).wait()
            pltpu.make_async_copy(dst_hbm.at[pl.ds(off, KB)], dstA,
                                  semA).wait()
            pltpu.make_async_copy(ea_hbm.at[pl.ds(off, KB)], eaA,
                                  semA).wait()
            pltpu.make_async_copy(se_hbm.at[pl.ds(off, KB)], seA,
                                  semA).wait()

        issue_a(ebase)
        for gi in range(KB // 16):
            sidxA[pl.ds(gi * 16, 16)] = jnp.full((16,), HALF, jnp.int32)
            sidxB[pl.ds(gi * 16, 16)] = jnp.full((16,), HALF, jnp.int32)
        pltpu.async_copy(outA, accum.at[sidxA], semSA, add=True)
        pltpu.async_copy(outB, accum.at[sidxB], semSB, add=True)

        def it(i, carry):
            offA = ebase + i * (2 * KB)
            offB = offA + KB
            b1 = pltpu.async_copy(src_hbm.at[pl.ds(offB, KB)], srcB, semB)
            b2 = pltpu.async_copy(dst_hbm.at[pl.ds(offB, KB)], dstB, semB)
            b3 = pltpu.async_copy(ea_hbm.at[pl.ds(offB, KB)], eaB, semB)
            b4 = pltpu.async_copy(se_hbm.at[pl.ds(offB, KB)], seB, semB)
            drain_a(offA)
            gA = pltpu.async_copy(g_hbm.at[srcA], growsA, semA)
            b1.wait(); b2.wait(); b3.wait(); b4.wait()
            gB = pltpu.async_copy(g_hbm.at[srcB], growsB, semB)
            gA.wait()
            pltpu.make_async_copy(outA, accum.at[pl.ds(0, KB)], semSA).wait()
            compute_half(growsA, eaA, seA, dstA, outA, sidxA,
                         range(KB // 16), 0)
            issue_a(offA + 2 * KB)
            pltpu.async_copy(outA, accum.at[sidxA], semSA, add=True)
            gB.wait()
            pltpu.make_async_copy(outB, accum.at[pl.ds(0, KB)], semSB).wait()
            compute_half(growsB, eaB, seB, dstB, outB, sidxB,
                         range(KB // 16), 0)
            pltpu.async_copy(outB, accum.at[sidxB], semSB, add=True)
            return carry

        lax.fori_loop(0, PAIRS, it, 0)
        offE = ebase + EPI_OFF
        drain_a(offE)
        gE = pltpu.async_copy(g_hbm.at[srcA], growsA, semA)
        gE.wait()
        pltpu.make_async_copy(outA, accum.at[pl.ds(0, KB)], semSA).wait()
        compute_half(growsA, eaA, seA, dstA, outA, sidx80,
                     range(EPI_SKIP // 16, KB // 16), EPI_SKIP // 16)
        if EPI_SKIP == 0:
            pltpu.sync_copy(outA, accum.at[sidx80], add=True)
        else:
            pltpu.sync_copy(outA.at[pl.ds(EPI_SKIP, KB - EPI_SKIP)],
                            accum.at[sidx80], add=True)
        pltpu.make_async_copy(outB, accum.at[pl.ds(0, KB)], semSB).wait()
        plsc.subcore_barrier()
        pltpu.sync_copy(accum.at[pl.ds(s * DSTRIPE, DSTRIPE)],
                        out_hbm.at[pl.ds(base + s * DSTRIPE, DSTRIPE)])

        @pl.when(s == 0)
        def _():
            pltpu.sync_copy(accum.at[pl.ds(NS * DSTRIPE, HALF - NS * DSTRIPE)],
                            out_hbm.at[pl.ds(base + NS * DSTRIPE,
                                             HALF - NS * DSTRIPE)])

    mesh = plsc.VectorSubcoreMesh(core_axis_name="c", subcore_axis_name="s")
    return pl.kernel(
        body,
        out_type=jax.ShapeDtypeStruct((N, roww), jnp.float32),
        mesh=mesh,
        compiler_params=pltpu.CompilerParams(use_tc_tiling_on_sc=False),
        scratch_types=[
            pltpu.VMEM_SHARED((ACC_ROWS, roww), jnp.float32),
            pltpu.VMEM((KB,), jnp.int32),
            pltpu.VMEM((KB,), jnp.int32),
            pltpu.VMEM((KB,), jnp.int32),
            pltpu.VMEM((KB,), jnp.int32),
            pltpu.VMEM((KB,), jnp.int32),
            pltpu.VMEM((KB,), jnp.int32),
            pltpu.VMEM((KB - EPI_SKIP,), jnp.int32),
            pltpu.VMEM((KB, gw), jnp.float32),
            pltpu.VMEM((KB, gw), jnp.float32),
            pltpu.VMEM((KB, EDIM), jnp.float32),
            pltpu.VMEM((KB, EDIM), jnp.float32),
            pltpu.VMEM((KB,), jnp.float32),
            pltpu.VMEM((KB,), jnp.float32),
            pltpu.VMEM((KB, roww), jnp.float32),
            pltpu.VMEM((KB, roww), jnp.float32),
            pltpu.SemaphoreType.DMA,
            pltpu.SemaphoreType.DMA,
            pltpu.SemaphoreType.DMA,
            pltpu.SemaphoreType.DMA,
        ],
    )


EB = 6400
NB = 2000


def _edge_prep_body(ea_ref, wv_ref, out_ref):
    out_ref[...] = jnp.dot(ea_ref[...], wv_ref[...],
                           preferred_element_type=jnp.float32)


def _edge_prep(edge_attr, wv):
    return pl.pallas_call(
        _edge_prep_body,
        grid=(E // EB,),
        in_specs=[
            pl.BlockSpec((EB, EDIM), lambda i: (i, 0)),
            pl.BlockSpec((EDIM, 2), lambda i: (0, 0)),
        ],
        out_specs=pl.BlockSpec((EB, 2), lambda i: (i, 0)),
        out_shape=jax.ShapeDtypeStruct((E, 2), jnp.float32),
    )(edge_attr, wv)


def _node_prep_body(h_ref, wn_ref, c_ref, out_ref):
    h = h_ref[...]
    col = jnp.dot(h, wn_ref[...], preferred_element_type=jnp.float32)
    pad = jnp.zeros((h.shape[0], 15), jnp.float32)
    out_ref[...] = jnp.concatenate([h, col + c_ref[0, 0], pad], axis=1)


def _node_prep(h, wn, c):
    din = h.shape[1]
    return pl.pallas_call(
        _node_prep_body,
        grid=(N // NB,),
        in_specs=[
            pl.BlockSpec((NB, din), lambda i: (i, 0)),
            pl.BlockSpec((din, 1), lambda i: (0, 0)),
            pl.BlockSpec((1, 1), lambda i: (0, 0)),
        ],
        out_specs=pl.BlockSpec((NB, din + 16), lambda i: (i, 0)),
        out_shape=jax.ShapeDtypeStruct((N, din + 16), jnp.float32),
    )(h, wn, c)


def _post_body(din, hout, next_cols, acc_ref, g_ref, mw_ref, ew_ref, bs_ref,
               sw_ref, sb_ref, wn_ref, c_ref, out_ref):
    acc = acc_ref[...]
    sh = acc[:, :din]
    sea = acc[:, din:din + EDIM]
    sex = acc[:, din + EDIM]
    cnt = acc[:, din + EDIM + 1]
    num = (jnp.dot(sh, mw_ref[...], preferred_element_type=jnp.float32)
           + jnp.dot(sea, ew_ref[...], preferred_element_type=jnp.float32)
           + sex[:, None] * bs_ref[...])
    agg = num / (sex + 1e-16)[:, None] / jnp.maximum(cnt, 1.0)[:, None]
    h = g_ref[...][:, :din]
    hn = jax.nn.relu(
        agg + jnp.dot(h, sw_ref[...], preferred_element_type=jnp.float32)
        + sb_ref[...])
    if next_cols:
        col = jnp.dot(hn, wn_ref[...], preferred_element_type=jnp.float32)
        pad = jnp.zeros((hn.shape[0], 15), jnp.float32)
        out_ref[...] = jnp.concatenate([hn, col + c_ref[0, 0], pad], axis=1)
    else:
        out_ref[...] = hn


def _post(acc, g, mw, ew, bsum, sw, sb, wn, c, next_cols):
    din = mw.shape[0]
    hout = mw.shape[1]
    ow = hout + (16 if next_cols else 0)
    return pl.pallas_call(
        functools.partial(_post_body, din, hout, next_cols),
        grid=(N // NB,),
        in_specs=[
            pl.BlockSpec((NB, acc.shape[1]), lambda i: (i, 0)),
            pl.BlockSpec((NB, g.shape[1]), lambda i: (i, 0)),
            pl.BlockSpec((din, hout), lambda i: (0, 0)),
            pl.BlockSpec((EDIM, hout), lambda i: (0, 0)),
            pl.BlockSpec((1, hout), lambda i: (0, 0)),
            pl.BlockSpec((din, hout), lambda i: (0, 0)),
            pl.BlockSpec((1, hout), lambda i: (0, 0)),
            pl.BlockSpec((hout, 1), lambda i: (0, 0)),
            pl.BlockSpec((1, 1), lambda i: (0, 0)),
        ],
        out_specs=pl.BlockSpec((NB, ow), lambda i: (i, 0)),
        out_shape=jax.ShapeDtypeStruct((N, ow), jnp.float32),
    )(acc, g, mw, ew, bsum, sw, sb, wn, c)


def _pool_body(h_ref, b_ref, w1_ref, b1_ref, w2_ref, b2_ref, out_ref, ps_ref):
    i = pl.program_id(0)

    @pl.when(i == 0)
    def _():
        ps_ref[...] = jnp.zeros_like(ps_ref)

    h = h_ref[...]
    bb = b_ref[...]
    onehot = (bb == lax.broadcasted_iota(jnp.int32, (1, NGRAPH), 1)
              ).astype(jnp.float32)
    hcat = jnp.concatenate(
        [h, jnp.ones((h.shape[0], 1), jnp.float32)], axis=1)
    ps_ref[...] += jnp.dot(onehot.T, hcat,
                           preferred_element_type=jnp.float32)
    ps = ps_ref[...]
    pooled = ps[:, :H2] / jnp.maximum(ps[:, H2:H2 + 1], 1.0)
    hidden = jax.nn.relu(
        jnp.dot(pooled, w1_ref[...], preferred_element_type=jnp.float32)
        + b1_ref[...])
    out_ref[...] = (jnp.dot(hidden, w2_ref[...],
                            preferred_element_type=jnp.float32) + b2_ref[...])


def _pool_cls(h, batch2d, w1, b1, w2, b2):
    return pl.pallas_call(
        _pool_body,
        grid=(N // NB,),
        in_specs=[
            pl.BlockSpec((NB, H2), lambda i: (i, 0)),
            pl.BlockSpec((NB, 1), lambda i: (i, 0)),
            pl.BlockSpec((H2, H2), lambda i: (0, 0)),
            pl.BlockSpec((1, H2), lambda i: (0, 0)),
            pl.BlockSpec((H2, OUT), lambda i: (0, 0)),
            pl.BlockSpec((1, OUT), lambda i: (0, 0)),
        ],
        out_specs=pl.BlockSpec((NGRAPH, OUT), lambda i: (0, 0)),
        out_shape=jax.ShapeDtypeStruct((NGRAPH, OUT), jnp.float32),
        scratch_shapes=[pltpu.VMEM((NGRAPH, H2 + 1), jnp.float32)],
    )(h, batch2d, w1, b1, w2, b2)


def kernel(x, edge_index, edge_attr, batchh, emb,
           l1_msg_w, l1_msg_b, l1_self_w, l1_self_b, l1_edge_w, l1_edge_b,
           l1_att,
           l2_msg_w, l2_msg_b, l2_self_w, l2_self_b, l2_edge_w, l2_edge_b,
           l2_att,
           cls_w1, cls_b1, cls_w2, cls_b2):
    h0 = jnp.take(emb, x, axis=0)
    srcs = edge_index[0]
    dsts = edge_index[1]

    a1 = l1_att[0]
    a2 = l2_att[0]
    wv = jnp.stack([l1_edge_w @ a1, l2_edge_w @ a2], axis=1)
    wn1 = (l1_msg_w @ a1)[:, None]
    c1 = jnp.reshape((l1_msg_b + l1_edge_b) @ a1, (1, 1))
    wn2 = (l2_msg_w @ a2)[:, None]
    c2 = jnp.reshape((l2_msg_b + l2_edge_b) @ a2, (1, 1))
    bs1 = (l1_msg_b + l1_edge_b)[None, :]
    bs2 = (l2_msg_b + l2_edge_b)[None, :]

    se_both = _edge_prep(edge_attr, wv)
    se1 = se_both[:, 0] + 0.0
    se2 = se_both[:, 1] + 0.0

    g1 = _node_prep(h0, wn1, c1)
    z1 = jnp.zeros((ZSTRIPE, EMB + 32), jnp.float32)
    acc1 = _sc_layer(EMB)(g1, srcs, dsts, edge_attr, se1, z1)
    g2 = _post(acc1, g1, l1_msg_w, l1_edge_w, bs1, l1_self_w,
               l1_self_b[None, :], wn2, c2, True)

    z2 = jnp.zeros((ZSTRIPE, H1 + 32), jnp.float32)
    acc2 = _sc_layer(H1)(g2, srcs, dsts, edge_attr, se2, z2)
    h2 = _post(acc2, g2, l2_msg_w, l2_edge_w, bs2, l2_self_w,
               l2_self_b[None, :], wn2, c2, False)

    return _pool_cls(h2, batchh[:, None], cls_w1, cls_b1[None, :],
                     cls_w2, cls_b2[None, :])
